# baseline (speedup 1.0000x reference)
import functools

import jax
import jax.numpy as jnp
from jax import lax
from jax.experimental import pallas as pl
from jax.experimental.pallas import tpu as pltpu
from jax.experimental.pallas import tpu_sc as plsc

_N = 4194304
_SPLIT = 2097152
_B_N = _N - _SPLIT

_BLK_TC = 262144
_GRID_A = _SPLIT // _BLK_TC
_GRID_B = _B_N // _BLK_TC

_NC = 2
_NS = 16
_NW = _NC * _NS
_CHUNK = _B_N // _NW
_BLK = 16384
_NBLK = _CHUNK // _BLK
_LANES = 16


_mesh = plsc.VectorSubcoreMesh(core_axis_name="c", subcore_axis_name="s")


@functools.partial(
    pl.kernel,
    mesh=_mesh,
    out_type=jax.ShapeDtypeStruct((_B_N,), jnp.int8),
    scratch_types=[
        pltpu.VMEM((_BLK,), jnp.int32),
        pltpu.VMEM((_BLK,), jnp.int32),
        pltpu.VMEM((_BLK,), jnp.int8),
        pltpu.VMEM((_BLK,), jnp.int8),
        pltpu.SemaphoreType.DMA,
        pltpu.SemaphoreType.DMA,
        pltpu.SemaphoreType.DMA,
        pltpu.SemaphoreType.DMA,
    ],
    compiler_params=pltpu.CompilerParams(needs_layout_passes=False),
)
def _pack_sc(idx_hbm, pk_hbm, idx_v0, idx_v1, pk_v0, pk_v1,
             in_s0, in_s1, out_s0, out_s1):
    wid = lax.axis_index("s") * _NC + lax.axis_index("c")
    base = wid * _CHUNK

    iota4 = lax.iota(jnp.int32, _LANES) * 4
    g0 = iota4
    g1 = iota4 + 2
    g2 = iota4 + 1
    g3 = iota4 + 3

    idx_bufs = (idx_v0, idx_v1)
    pk_bufs = (pk_v0, pk_v1)
    in_sems = (in_s0, in_s1)
    out_sems = (out_s0, out_s1)

    in_copies = {}
    out_copies = {}

    def start_in(b):
        in_copies[b] = pltpu.async_copy(
            idx_hbm.at[pl.ds(_SPLIT + base + b * _BLK, _BLK)],
            idx_bufs[b % 2], in_sems[b % 2])

    start_in(0)
    for b in range(_NBLK):
        ib = idx_bufs[b % 2]
        ob = pk_bufs[b % 2]
        in_copies.pop(b).wait()
        if b + 1 < _NBLK:
            start_in(b + 1)
        if b >= 2:
            out_copies.pop(b - 2).wait()

        @plsc.parallel_loop(0, _BLK, 4 * _LANES, unroll=4)
        def body(s):
            s = pl.multiple_of(s, 4 * _LANES)
            u = plsc.load_gather(ib, [g0 + s])
            v = plsc.load_gather(ib, [g1 + s])
            u2 = plsc.load_gather(ib, [g2 + s])
            v2 = plsc.load_gather(ib, [g3 + s])
            x = plsc.pack(u, v, format=plsc.PackFormat.INTERLEAVED)
            y = plsc.pack(u2, v2, format=plsc.PackFormat.INTERLEAVED)
            z = plsc.pack(x, y, format=plsc.PackFormat.INTERLEAVED,
                          preferred_element_type=jnp.int8)
            ob[pl.ds(s, 4 * _LANES)] = z

        out_copies[b] = pltpu.async_copy(
            ob, pk_hbm.at[pl.ds(base + b * _BLK, _BLK)], out_sems[b % 2])

    out_copies.pop(_NBLK - 2).wait()
    out_copies.pop(_NBLK - 1).wait()


def _tc_direct_body(tab_ref, idx_ref, out_ref):
    i = idx_ref[...]
    lo = jnp.where(i == 0, tab_ref[0], tab_ref[1])
    hi = jnp.where(i == 2, tab_ref[2], tab_ref[3])
    out_ref[...] = jnp.where(i < 2, lo, hi)


_tc_direct = pl.pallas_call(
    _tc_direct_body,
    grid=(_GRID_A,),
    in_specs=[
        pl.BlockSpec(memory_space=pltpu.SMEM),
        pl.BlockSpec((_BLK_TC,), lambda i: (i,)),
    ],
    out_specs=pl.BlockSpec((_BLK_TC,), lambda i: (i,)),
    out_shape=jax.ShapeDtypeStruct((_N,), jnp.float32),
    compiler_params=pltpu.CompilerParams(
        dimension_semantics=("arbitrary",),
    ),
)


def _tc_decode_body(tab_ref, pk_ref, _, out_ref):
    i = pk_ref[...].astype(jnp.int32)
    lo = jnp.where(i == 0, tab_ref[0], tab_ref[1])
    hi = jnp.where(i == 2, tab_ref[2], tab_ref[3])
    out_ref[...] = jnp.where(i < 2, lo, hi)


_tc_decode = pl.pallas_call(
    _tc_decode_body,
    grid=(_GRID_B,),
    in_specs=[
        pl.BlockSpec(memory_space=pltpu.SMEM),
        pl.BlockSpec((_BLK_TC,), lambda i: (i,)),
        pl.BlockSpec(memory_space=pl.ANY),
    ],
    out_specs=pl.BlockSpec((_BLK_TC,), lambda i: (i + _GRID_A,)),
    out_shape=jax.ShapeDtypeStruct((_N,), jnp.float32),
    input_output_aliases={2: 0},
    compiler_params=pltpu.CompilerParams(
        dimension_semantics=("arbitrary",),
    ),
)


def kernel(supervision_weight, index, dummy, bin_num_examples):
    packed = _pack_sc(index)
    first = _tc_direct(bin_num_examples, index)
    return _tc_decode(bin_num_examples, packed, first)


# speedup vs baseline: 1.3047x; 1.3047x over previous
import functools

import jax
import jax.numpy as jnp
from jax import lax
from jax.experimental import pallas as pl
from jax.experimental.pallas import tpu as pltpu
from jax.experimental.pallas import tpu_sc as plsc

_N = 4194304
_SPLIT = 2097152
_B_N = _N - _SPLIT
_NSTREAM = 16
_SB = _B_N // _NSTREAM
_NWORDS = _B_N // _NSTREAM

_BLK_TC = 262144
_GRID_A = _SPLIT // _BLK_TC

_NC = 2
_NS = 16
_NW = _NC * _NS
_WCHUNK = _NWORDS // _NW
_WBLK = 2048
_NBLK = _WCHUNK // _WBLK
_LANES = 16


_mesh = plsc.VectorSubcoreMesh(core_axis_name="c", subcore_axis_name="s")


@functools.partial(
    pl.kernel,
    mesh=_mesh,
    out_type=jax.ShapeDtypeStruct((_NWORDS,), jnp.int32),
    scratch_types=[
        pltpu.VMEM((_NSTREAM, _WBLK), jnp.int32),
        pltpu.VMEM((_NSTREAM, _WBLK), jnp.int32),
        pltpu.VMEM((_WBLK,), jnp.int32),
        pltpu.VMEM((_WBLK,), jnp.int32),
        pltpu.SemaphoreType.DMA,
        pltpu.SemaphoreType.DMA,
        pltpu.SemaphoreType.DMA,
        pltpu.SemaphoreType.DMA,
    ],
    compiler_params=pltpu.CompilerParams(needs_layout_passes=False),
)
def _pack_sc(idx_hbm, pk_hbm, in_v0, in_v1, pk_v0, pk_v1,
             in_s0, in_s1, out_s0, out_s1):
    wid = lax.axis_index("s") * _NC + lax.axis_index("c")
    base = wid * _WCHUNK

    in_bufs = (in_v0, in_v1)
    pk_bufs = (pk_v0, pk_v1)
    in_sems = (in_s0, in_s1)
    out_sems = (out_s0, out_s1)

    in_copies = {}
    out_copies = {}

    def start_in(b):
        cps = []
        for k in range(_NSTREAM):
            cps.append(pltpu.async_copy(
                idx_hbm.at[pl.ds(_SPLIT + k * _SB + base + b * _WBLK, _WBLK)],
                in_bufs[b % 2].at[k], in_sems[b % 2]))
        in_copies[b] = cps

    start_in(0)
    for b in range(_NBLK):
        ib = in_bufs[b % 2]
        ob = pk_bufs[b % 2]
        for cp in in_copies.pop(b):
            cp.wait()
        if b + 1 < _NBLK:
            start_in(b + 1)
        if b >= 2:
            out_copies.pop(b - 2).wait()

        @plsc.parallel_loop(0, _WBLK, _LANES, unroll=2)
        def body(s):
            s = pl.multiple_of(s, _LANES)
            w = ib[0, pl.ds(s, _LANES)]
            for k in range(1, _NSTREAM):
                w = w | (ib[k, pl.ds(s, _LANES)] << (2 * k))
            ob[pl.ds(s, _LANES)] = w

        out_copies[b] = pltpu.async_copy(
            ob, pk_hbm.at[pl.ds(base + b * _WBLK, _WBLK)], out_sems[b % 2])

    out_copies.pop(_NBLK - 2).wait()
    out_copies.pop(_NBLK - 1).wait()


def _tc_direct_body(tab_ref, idx_ref, out_ref):
    i = idx_ref[...]
    lo = jnp.where(i == 0, tab_ref[0], tab_ref[1])
    hi = jnp.where(i == 2, tab_ref[2], tab_ref[3])
    out_ref[...] = jnp.where(i < 2, lo, hi)


_tc_direct = pl.pallas_call(
    _tc_direct_body,
    grid=(_GRID_A,),
    in_specs=[
        pl.BlockSpec(memory_space=pltpu.SMEM),
        pl.BlockSpec((_BLK_TC,), lambda i: (i,)),
    ],
    out_specs=pl.BlockSpec((_BLK_TC,), lambda i: (i,)),
    out_shape=jax.ShapeDtypeStruct((_N,), jnp.float32),
    compiler_params=pltpu.CompilerParams(
        dimension_semantics=("arbitrary",),
    ),
)


def _tc_decode_body(tab_ref, pk_ref, _, out_ref):
    j = pl.program_id(0)
    i = (pk_ref[...] >> (2 * j)) & 3
    lo = jnp.where(i == 0, tab_ref[0], tab_ref[1])
    hi = jnp.where(i == 2, tab_ref[2], tab_ref[3])
    out_ref[...] = jnp.where(i < 2, lo, hi)


_tc_decode = pl.pallas_call(
    _tc_decode_body,
    grid=(_NSTREAM,),
    in_specs=[
        pl.BlockSpec(memory_space=pltpu.SMEM),
        pl.BlockSpec((_NWORDS,), lambda j: (0,)),
        pl.BlockSpec(memory_space=pl.ANY),
    ],
    out_specs=pl.BlockSpec((_SB,), lambda j: (j + _SPLIT // _SB,)),
    out_shape=jax.ShapeDtypeStruct((_N,), jnp.float32),
    input_output_aliases={2: 0},
    compiler_params=pltpu.CompilerParams(
        dimension_semantics=("arbitrary",),
    ),
)


def kernel(supervision_weight, index, dummy, bin_num_examples):
    packed = _pack_sc(index)
    first = _tc_direct(bin_num_examples, index)
    return _tc_decode(bin_num_examples, packed, first)


# speedup vs baseline: 1.4428x; 1.1058x over previous
import functools

import jax
import jax.numpy as jnp
from jax import lax
from jax.experimental import pallas as pl
from jax.experimental.pallas import tpu as pltpu
from jax.experimental.pallas import tpu_sc as plsc

_N = 4194304
_NC = 2
_NS = 16
_NW = _NC * _NS
_CHUNK = _N // _NW
_BLK = 16384
_NBLK = _CHUNK // _BLK
_LANES = 16

_mesh = plsc.VectorSubcoreMesh(core_axis_name="c", subcore_axis_name="s")


@functools.partial(
    pl.kernel,
    mesh=_mesh,
    out_type=jax.ShapeDtypeStruct((_N,), jnp.float32),
    scratch_types=[
        pltpu.VMEM((_LANES,), jnp.float32),
        pltpu.VMEM((_BLK,), jnp.int32),
        pltpu.VMEM((_BLK,), jnp.int32),
        pltpu.VMEM((_BLK,), jnp.float32),
        pltpu.VMEM((_BLK,), jnp.float32),
        pltpu.SemaphoreType.DMA,
        pltpu.SemaphoreType.DMA,
        pltpu.SemaphoreType.DMA,
        pltpu.SemaphoreType.DMA,
    ],
)
def _gather_sc(table_hbm, idx_hbm, out_hbm, table_v,
               idx_v0, idx_v1, out_v0, out_v1,
               in_s0, in_s1, out_s0, out_s1):
    wid = lax.axis_index("s") * _NC + lax.axis_index("c")
    base = wid * _CHUNK
    pltpu.sync_copy(table_hbm, table_v)
    tab = table_v[...]

    idx_bufs = (idx_v0, idx_v1)
    out_bufs = (out_v0, out_v1)
    in_sems = (in_s0, in_s1)
    out_sems = (out_s0, out_s1)

    in_copies = {}
    out_copies = {}

    def start_in(b):
        in_copies[b] = pltpu.async_copy(
            idx_hbm.at[pl.ds(base + b * _BLK, _BLK)],
            idx_bufs[b % 2], in_sems[b % 2])

    start_in(0)
    for b in range(_NBLK):
        ib = idx_bufs[b % 2]
        ob = out_bufs[b % 2]
        in_copies.pop(b).wait()
        if b + 1 < _NBLK:
            start_in(b + 1)
        if b >= 2:
            out_copies.pop(b - 2).wait()

        @plsc.parallel_loop(0, _BLK, _LANES, unroll=8)
        def body(s):
            s = pl.multiple_of(s, _LANES)
            idx = ib[pl.ds(s, _LANES)]
            ob[pl.ds(s, _LANES)] = tab.at[idx].get(mode="promise_in_bounds")

        out_copies[b] = pltpu.async_copy(
            ob, out_hbm.at[pl.ds(base + b * _BLK, _BLK)], out_sems[b % 2])

    out_copies.pop(_NBLK - 2).wait()
    out_copies.pop(_NBLK - 1).wait()


def kernel(supervision_weight, index, dummy, bin_num_examples):
    table = jnp.pad(bin_num_examples, (0, _LANES - bin_num_examples.shape[0]))
    return _gather_sc(table, index)


# speedup vs baseline: 1.4679x; 1.0174x over previous
import functools

import jax
import jax.numpy as jnp
from jax import lax
from jax.experimental import pallas as pl
from jax.experimental.pallas import tpu as pltpu
from jax.experimental.pallas import tpu_sc as plsc

_N = 4194304
_NC = 2
_NS = 16
_NW = _NC * _NS
_CHUNK = _N // _NW
_BLK = 16384
_NBLK = _CHUNK // _BLK
_LANES = 16

_mesh = plsc.VectorSubcoreMesh(core_axis_name="c", subcore_axis_name="s")


@functools.partial(
    pl.kernel,
    mesh=_mesh,
    out_type=jax.ShapeDtypeStruct((_N,), jnp.float32),
    scratch_types=[
        pltpu.VMEM((_LANES,), jnp.float32),
        pltpu.VMEM((_BLK,), jnp.int32),
        pltpu.VMEM((_BLK,), jnp.int32),
        pltpu.VMEM((_BLK,), jnp.float32),
        pltpu.VMEM((_BLK,), jnp.float32),
        pltpu.SemaphoreType.DMA,
        pltpu.SemaphoreType.DMA,
        pltpu.SemaphoreType.DMA,
        pltpu.SemaphoreType.DMA,
    ],
)
def _gather_sc(table_hbm, idx_hbm, out_hbm, table_v,
               idx_v0, idx_v1, out_v0, out_v1,
               in_s0, in_s1, out_s0, out_s1):
    wid = lax.axis_index("s") * _NC + lax.axis_index("c")
    base = wid * _CHUNK
    pltpu.sync_copy(table_hbm, table_v.at[pl.ds(0, 4)])
    tab = table_v[...]

    idx_bufs = (idx_v0, idx_v1)
    out_bufs = (out_v0, out_v1)
    in_sems = (in_s0, in_s1)
    out_sems = (out_s0, out_s1)

    in_copies = {}
    out_copies = {}

    def start_in(b):
        in_copies[b] = pltpu.async_copy(
            idx_hbm.at[pl.ds(base + b * _BLK, _BLK)],
            idx_bufs[b % 2], in_sems[b % 2])

    start_in(0)
    for b in range(_NBLK):
        ib = idx_bufs[b % 2]
        ob = out_bufs[b % 2]
        in_copies.pop(b).wait()
        if b + 1 < _NBLK:
            start_in(b + 1)
        if b >= 2:
            out_copies.pop(b - 2).wait()

        @plsc.parallel_loop(0, _BLK, _LANES, unroll=8)
        def body(s):
            s = pl.multiple_of(s, _LANES)
            idx = ib[pl.ds(s, _LANES)]
            ob[pl.ds(s, _LANES)] = tab.at[idx].get(mode="promise_in_bounds")

        out_copies[b] = pltpu.async_copy(
            ob, out_hbm.at[pl.ds(base + b * _BLK, _BLK)], out_sems[b % 2])

    out_copies.pop(_NBLK - 2).wait()
    out_copies.pop(_NBLK - 1).wait()


def kernel(supervision_weight, index, dummy, bin_num_examples):
    return _gather_sc(bin_num_examples, index)
